# Initial kernel scaffold; baseline (speedup 1.0000x reference)
import jax
import jax.numpy as jnp
from jax.experimental import pallas as pl


def kernel(local_entity, batch_heads, batch_rels, batch_tails, batch_ids, fact_ids, weight_list, rel_features, W, b):
    raise NotImplementedError("write your pallas kernel here")


# speedup vs baseline: 1.4820x; 1.4820x over previous
import functools

import jax
import jax.numpy as jnp
from jax import lax
from jax.experimental import pallas as pl
from jax.experimental.pallas import tpu as pltpu
from jax.experimental.pallas import tpu_sc as plsc

_NC = 2
_NS = 16
_PASSES = 4
_LANES = 16


def _sc_histogram(heads, tails, rels, E, R):
    NF = heads.shape[0]
    FS = NF // _NS
    CE = E // (_NC * _PASSES)
    CW = CE * R
    TRASH = CW
    CHUNKS = FS // _LANES
    NI = FS
    ZB = 8000
    ZSL = CW // _NS

    mesh = plsc.VectorSubcoreMesh(core_axis_name="c", subcore_axis_name="s")

    @functools.partial(
        pl.kernel,
        out_type=jax.ShapeDtypeStruct((_NC * _PASSES * CW,), jnp.float32),
        mesh=mesh,
        scratch_types=[
            pltpu.VMEM((FS,), jnp.int32),
            pltpu.VMEM((FS,), jnp.int32),
            pltpu.VMEM((FS,), jnp.int32),
            pltpu.VMEM((NI,), jnp.int32),
            pltpu.VMEM((NI,), jnp.int32),
            pltpu.VMEM((NI,), jnp.float32),
            pltpu.VMEM((ZB,), jnp.float32),
            pltpu.VMEM_SHARED((CW + 8,), jnp.float32),
        ],
    )
    def hist_kernel(heads_hbm, tails_hbm, rels_hbm, out_hbm,
                    heads_v, tails_v, rels_v, idx_h, idx_t, ones_v,
                    zeros_v, acc):
        c = lax.axis_index("c")
        s = lax.axis_index("s")
        base_f = s * FS
        pltpu.sync_copy(heads_hbm.at[pl.ds(base_f, FS)], heads_v)
        pltpu.sync_copy(tails_hbm.at[pl.ds(base_f, FS)], tails_v)
        pltpu.sync_copy(rels_hbm.at[pl.ds(base_f, FS)], rels_v)

        trash_vec = jnp.full((_LANES,), TRASH, jnp.int32)
        one_vec = jnp.full((_LANES,), 1.0, jnp.float32)
        zero_vec = jnp.zeros((_LANES,), jnp.float32)

        def init_body(i, carry):
            ones_v[pl.ds(i * _LANES, _LANES)] = one_vec
            return carry

        lax.fori_loop(0, NI // _LANES, init_body, 0)

        def zinit_body(i, carry):
            zeros_v[pl.ds(i * _LANES, _LANES)] = zero_vec
            return carry

        lax.fori_loop(0, ZB // _LANES, zinit_body, 0)

        def one_pass(p, carry):
            base_w = (p * _NC + c) * CW

            def zero_acc(k, kc):
                pltpu.sync_copy(zeros_v,
                                acc.at[pl.ds(s * ZSL + k * ZB, ZB)])
                return kc

            lax.fori_loop(0, ZSL // ZB, zero_acc, 0)
            plsc.subcore_barrier()

            def compute(i, ic):
                e_h = heads_v[pl.ds(i * _LANES, _LANES)]
                e_t = tails_v[pl.ds(i * _LANES, _LANES)]
                r = rels_v[pl.ds(i * _LANES, _LANES)]
                g_h = e_h * R + r - base_w
                g_t = e_t * R + r - base_w
                m_h = (g_h >= 0) & (g_h < CW)
                m_t = (g_t >= 0) & (g_t < CW)
                idx_h[pl.ds(i * _LANES, _LANES)] = jnp.where(m_h, g_h, TRASH)
                idx_t[pl.ds(i * _LANES, _LANES)] = jnp.where(m_t, g_t, TRASH)
                return ic

            lax.fori_loop(0, CHUNKS, compute, 0)

            def serial_scatter(k, kc):
                @pl.when(s == k)
                def _():
                    pltpu.sync_copy(ones_v, acc.at[idx_h], add=True)
                    pltpu.sync_copy(ones_v, acc.at[idx_t], add=True)

                plsc.subcore_barrier()
                return kc

            lax.fori_loop(0, _NS, serial_scatter, 0)
            plsc.subcore_barrier()

            @pl.when(s == 0)
            def _():
                pltpu.sync_copy(acc.at[pl.ds(0, CW)],
                                out_hbm.at[pl.ds(base_w, CW)])

            plsc.subcore_barrier()
            return carry

        lax.fori_loop(0, _PASSES, one_pass, 0)

    return hist_kernel(heads, tails, rels)


def _tc_matmul(cmat, rf, w, b2):
    E, R = cmat.shape
    D = w.shape[0]
    BLK = 1000

    def body(c_ref, rf_ref, w_ref, b_ref, out_ref, relv_ref):
        @pl.when(pl.program_id(0) == 0)
        def _():
            relv_ref[...] = lax.dot_general(
                rf_ref[...], w_ref[...], (((1,), (1,)), ((), ())),
                preferred_element_type=jnp.float32) + b_ref[...]

        out_ref[...] = jnp.maximum(
            lax.dot_general(c_ref[...], relv_ref[...],
                            (((1,), (0,)), ((), ())),
                            preferred_element_type=jnp.float32), 0.0)

    return pl.pallas_call(
        body,
        grid=(E // BLK,),
        in_specs=[
            pl.BlockSpec((BLK, R), lambda i: (i, 0)),
            pl.BlockSpec((R, D), lambda i: (0, 0)),
            pl.BlockSpec((D, D), lambda i: (0, 0)),
            pl.BlockSpec((1, D), lambda i: (0, 0)),
        ],
        out_specs=pl.BlockSpec((BLK, D), lambda i: (i, 0)),
        out_shape=jax.ShapeDtypeStruct((E, D), jnp.float32),
        scratch_shapes=[pltpu.VMEM((R, D), jnp.float32)],
    )(cmat, rf, w, b2)


def kernel(local_entity, batch_heads, batch_rels, batch_tails, batch_ids,
           fact_ids, weight_list, rel_features, W, b):
    B, M = local_entity.shape
    E = B * M
    R = rel_features.shape[0]
    D = W.shape[0]
    hist = _sc_histogram(batch_heads, batch_tails, batch_rels, E, R)
    cmat = hist.reshape(E, R)
    out = _tc_matmul(cmat, rel_features, W, b.reshape(1, D))
    return out.reshape(B, M, D)


# speedup vs baseline: 4.2315x; 2.8553x over previous
import functools

import jax
import jax.numpy as jnp
from jax import lax
from jax.experimental import pallas as pl
from jax.experimental.pallas import tpu as pltpu
from jax.experimental.pallas import tpu_sc as plsc

_NC = 2
_NS = 16
_PASSES = 4
_LANES = 16
_ZB = 8000


def _sc_histogram(ents, rels, E, R):
    NF = ents.shape[0] // _NC
    FS = NF // _NS
    CW = E * R // _PASSES
    TRASH = CW
    CHUNKS = FS // _LANES
    ZSL = CW // _NS

    mesh = plsc.VectorSubcoreMesh(core_axis_name="c", subcore_axis_name="s")

    @functools.partial(
        pl.kernel,
        out_type=jax.ShapeDtypeStruct((_NC * E * R,), jnp.int32),
        mesh=mesh,
        scratch_types=[
            pltpu.VMEM((FS,), jnp.int32),
            pltpu.VMEM((FS,), jnp.int32),
            pltpu.VMEM((FS,), jnp.int32),
            pltpu.VMEM((FS,), jnp.int32),
            pltpu.VMEM((_ZB,), jnp.int32),
            pltpu.VMEM_SHARED((CW + 8,), jnp.int32),
        ],
    )
    def hist_kernel(ents_hbm, rels_hbm, out_hbm,
                    ents_v, rels_v, idx_v, ones_v, zeros_v, acc):
        c = lax.axis_index("c")
        s = lax.axis_index("s")
        base_f = s * FS
        pltpu.sync_copy(ents_hbm.at[pl.ds(c * NF + base_f, FS)], ents_v)
        pltpu.sync_copy(rels_hbm.at[pl.ds(base_f, FS)], rels_v)

        one_vec = jnp.full((_LANES,), 1, jnp.int32)
        zero_vec = jnp.zeros((_LANES,), jnp.int32)

        def init_body(i, carry):
            ones_v[pl.ds(i * _LANES, _LANES)] = one_vec
            return carry

        lax.fori_loop(0, FS // _LANES, init_body, 0)

        def zinit_body(i, carry):
            zeros_v[pl.ds(i * _LANES, _LANES)] = zero_vec
            return carry

        lax.fori_loop(0, _ZB // _LANES, zinit_body, 0)

        def one_pass(p, carry):
            base_w = p * CW

            def zero_acc(k, kc):
                pltpu.sync_copy(zeros_v, acc.at[pl.ds(s * ZSL + k * _ZB, _ZB)])
                return kc

            lax.fori_loop(0, ZSL // _ZB, zero_acc, 0)
            plsc.subcore_barrier()

            def compute(i, ic):
                e = ents_v[pl.ds(i * _LANES, _LANES)]
                r = rels_v[pl.ds(i * _LANES, _LANES)]
                g = e * R + r - base_w
                m = (g >= 0) & (g < CW)
                idx_v[pl.ds(i * _LANES, _LANES)] = jnp.where(m, g, TRASH)
                return ic

            lax.fori_loop(0, CHUNKS, compute, 0)

            def serial_scatter(k, kc):
                @pl.when(s == k)
                def _():
                    pltpu.sync_copy(ones_v, acc.at[idx_v], add=True)

                plsc.subcore_barrier()
                return kc

            lax.fori_loop(0, _NS, serial_scatter, 0)

            @pl.when(s == 0)
            def _():
                pltpu.sync_copy(acc.at[pl.ds(0, CW)],
                                out_hbm.at[pl.ds(c * E * R + base_w, CW)])

            plsc.subcore_barrier()
            return carry

        lax.fori_loop(0, _PASSES, one_pass, 0)

    return hist_kernel(ents, rels)


def _tc_matmul(ch, ct, rf, w, b2):
    E, R = ch.shape
    D = w.shape[0]
    BLK = 1000

    def body(ch_ref, ct_ref, rf_ref, w_ref, b_ref, out_ref, relv_ref):
        @pl.when(pl.program_id(0) == 0)
        def _():
            relv_ref[...] = lax.dot_general(
                rf_ref[...], w_ref[...], (((1,), (1,)), ((), ())),
                preferred_element_type=jnp.float32) + b_ref[...]

        cmat = (ch_ref[...] + ct_ref[...]).astype(jnp.float32)
        out_ref[...] = jnp.maximum(
            lax.dot_general(cmat, relv_ref[...], (((1,), (0,)), ((), ())),
                            preferred_element_type=jnp.float32), 0.0)

    return pl.pallas_call(
        body,
        grid=(E // BLK,),
        in_specs=[
            pl.BlockSpec((BLK, R), lambda i: (i, 0)),
            pl.BlockSpec((BLK, R), lambda i: (i, 0)),
            pl.BlockSpec((R, D), lambda i: (0, 0)),
            pl.BlockSpec((D, D), lambda i: (0, 0)),
            pl.BlockSpec((1, D), lambda i: (0, 0)),
        ],
        out_specs=pl.BlockSpec((BLK, D), lambda i: (i, 0)),
        out_shape=jax.ShapeDtypeStruct((E, D), jnp.float32),
        scratch_shapes=[pltpu.VMEM((R, D), jnp.float32)],
    )(ch, ct, rf, w, b2)


def kernel(local_entity, batch_heads, batch_rels, batch_tails, batch_ids,
           fact_ids, weight_list, rel_features, W, b):
    B, M = local_entity.shape
    E = B * M
    R = rel_features.shape[0]
    D = W.shape[0]
    ents = jnp.concatenate([batch_heads, batch_tails])
    hist = _sc_histogram(ents, batch_rels, E, R)
    ch = hist[:E * R].reshape(E, R)
    ct = hist[E * R:].reshape(E, R)
    out = _tc_matmul(ch, ct, rel_features, W, b.reshape(1, D))
    return out.reshape(B, M, D)


# speedup vs baseline: 5.1646x; 1.2205x over previous
import functools

import jax
import jax.numpy as jnp
from jax import lax
from jax.experimental import pallas as pl
from jax.experimental.pallas import tpu as pltpu
from jax.experimental.pallas import tpu_sc as plsc

_NC = 2
_NS = 16
_PASSES = 4
_CW = 1280000
_LANES = 16
_ZB = 8000
_TZ = 16384


def _sc_histogram(ents, rels, E, R):
    NF = ents.shape[0] // _NC
    FS = NF // _NS
    CW = _CW
    PAD = _PASSES * CW
    CHUNKS = FS // _LANES
    ZSL = CW // _NS

    mesh = plsc.VectorSubcoreMesh(core_axis_name="c", subcore_axis_name="s")

    @functools.partial(
        pl.kernel,
        out_type=jax.ShapeDtypeStruct((_NC * PAD,), jnp.float32),
        mesh=mesh,
        scratch_types=[
            pltpu.VMEM((FS,), jnp.int32),
            pltpu.VMEM((FS,), jnp.int32),
            pltpu.VMEM((FS,), jnp.int32),
            pltpu.VMEM((FS,), jnp.float32),
            pltpu.VMEM((_ZB,), jnp.float32),
            pltpu.VMEM_SHARED((CW + _TZ,), jnp.float32),
        ],
    )
    def hist_kernel(ents_hbm, rels_hbm, out_hbm,
                    ents_v, rels_v, idx_v, ones_v, zeros_v, acc):
        c = lax.axis_index("c")
        s = lax.axis_index("s")
        base_f = s * FS
        pltpu.sync_copy(ents_hbm.at[pl.ds(c * NF + base_f, FS)], ents_v)
        pltpu.sync_copy(rels_hbm.at[pl.ds(base_f, FS)], rels_v)

        one_vec = jnp.full((_LANES,), 1.0, jnp.float32)
        zero_vec = jnp.zeros((_LANES,), jnp.float32)

        def init_body(i, carry):
            ones_v[pl.ds(i * _LANES, _LANES)] = one_vec
            return carry

        lax.fori_loop(0, FS // _LANES, init_body, 0)

        def zinit_body(i, carry):
            zeros_v[pl.ds(i * _LANES, _LANES)] = zero_vec
            return carry

        lax.fori_loop(0, _ZB // _LANES, zinit_body, 0)

        def one_pass(p, carry):
            base_w = p * CW

            def zero_acc(k, kc):
                pltpu.sync_copy(zeros_v, acc.at[pl.ds(s * ZSL + k * _ZB, _ZB)])
                return kc

            lax.fori_loop(0, ZSL // _ZB, zero_acc, 0)
            plsc.subcore_barrier()

            def compute(i, ic):
                e = ents_v[pl.ds(i * _LANES, _LANES)]
                r = rels_v[pl.ds(i * _LANES, _LANES)]
                raw = e * R + r
                g = raw - base_w
                m = (g >= 0) & (g < CW)
                trash = CW + (raw & (_TZ - 1))
                idx_v[pl.ds(i * _LANES, _LANES)] = jnp.where(m, g, trash)
                return ic

            lax.fori_loop(0, CHUNKS, compute, 0)

            def serial_scatter(k, kc):
                @pl.when(s == k)
                def _():
                    pltpu.sync_copy(ones_v, acc.at[idx_v], add=True)

                plsc.subcore_barrier()
                return kc

            lax.fori_loop(0, _NS, serial_scatter, 0)

            @pl.when(s == 0)
            def _():
                pltpu.sync_copy(acc.at[pl.ds(0, CW)],
                                out_hbm.at[pl.ds(c * PAD + base_w, CW)])

            plsc.subcore_barrier()
            return carry

        lax.fori_loop(0, _PASSES, one_pass, 0)

    return hist_kernel(ents, rels)


def _tc_matmul(ch, ct, rf, w, b2):
    E, R = ch.shape
    D = w.shape[0]
    BLK = 1000

    def body(ch_ref, ct_ref, rf_ref, w_ref, b_ref, out_ref, relv_ref):
        @pl.when(pl.program_id(0) == 0)
        def _():
            relv_ref[...] = lax.dot_general(
                rf_ref[...], w_ref[...], (((1,), (1,)), ((), ())),
                preferred_element_type=jnp.float32) + b_ref[...]

        cmat = ch_ref[...] + ct_ref[...]
        out_ref[...] = jnp.maximum(
            lax.dot_general(cmat, relv_ref[...], (((1,), (0,)), ((), ())),
                            preferred_element_type=jnp.float32), 0.0)

    return pl.pallas_call(
        body,
        grid=(E // BLK,),
        in_specs=[
            pl.BlockSpec((BLK, R), lambda i: (i, 0)),
            pl.BlockSpec((BLK, R), lambda i: (i, 0)),
            pl.BlockSpec((R, D), lambda i: (0, 0)),
            pl.BlockSpec((D, D), lambda i: (0, 0)),
            pl.BlockSpec((1, D), lambda i: (0, 0)),
        ],
        out_specs=pl.BlockSpec((BLK, D), lambda i: (i, 0)),
        out_shape=jax.ShapeDtypeStruct((E, D), jnp.float32),
        scratch_shapes=[pltpu.VMEM((R, D), jnp.float32)],
    )(ch, ct, rf, w, b2)


def kernel(local_entity, batch_heads, batch_rels, batch_tails, batch_ids,
           fact_ids, weight_list, rel_features, W, b):
    B, M = local_entity.shape
    E = B * M
    R = rel_features.shape[0]
    D = W.shape[0]
    ents = jnp.concatenate([batch_heads, batch_tails])
    hist = _sc_histogram(ents, batch_rels, E, R)
    pad = _PASSES * _CW
    ch = hist[:E * R].reshape(E, R)
    ct = hist[pad:pad + E * R].reshape(E, R)
    out = _tc_matmul(ch, ct, rel_features, W, b.reshape(1, D))
    return out.reshape(B, M, D)
